# baseline (speedup 1.0000x reference)
import dataclasses
import functools

import jax
import jax.numpy as jnp
from jax import lax
from jax.experimental import pallas as pl
from jax.experimental.pallas import tpu as pltpu
from jax.experimental.pallas import tpu_sc as plsc

N = 10000
D = 128
DFF = 256
OFF = 36
E = 320000

NC = 2
NS = 16
NW = NC * NS
EBLK = 128
K = 80
KD = K
E_PAD = NW * K * EBLK
N_PAD = 10240
NWIN = 3
WROWS = 3456
DUMMY = N
NBUF = 4

BR = 512


def _mesh():
    return plsc.VectorSubcoreMesh(core_axis_name="c", subcore_axis_name="s")


def _sc_params():
    cp = pltpu.CompilerParams()
    if "needs_layout_passes" in pltpu.CompilerParams.__dataclass_fields__:
        cp = dataclasses.replace(cp, needs_layout_passes=False)
    return cp


def _zero_fill(buf, rows, width, value=0.0):
    vec = jnp.full((16,), value, jnp.float32)

    @pl.loop(0, rows)
    def _(r):
        @pl.loop(0, width, step=16)
        def _(j):
            buf[r, pl.ds(j, 16)] = vec


@functools.cache
def _sc_segment_sum_kernel():
    @functools.partial(
        pl.kernel,
        out_type=jax.ShapeDtypeStruct((NC, N_PAD, D), jnp.float32),
        mesh=_mesh(),
        scratch_types=[
            pltpu.VMEM((K, EBLK), jnp.int32),
            pltpu.VMEM((K, EBLK), jnp.int32),
            pltpu.VMEM((K, EBLK), jnp.int32),
            pltpu.VMEM((NBUF * EBLK, D), jnp.float32),
            pltpu.VMEM_SHARED((WROWS + 8, D), jnp.float32),
            pltpu.SemaphoreType.DMA,
            pltpu.SemaphoreType.DMA,
        ],
        compiler_params=_sc_params(),
    )
    def k(vals_hbm, idx_hbm, out_hbm,
          idx_v, src_v, dst_l, bufs, acc, sem_g, sem_s):
        c = lax.axis_index("c")
        s = lax.axis_index("s")
        wid = s * NC + c
        pltpu.sync_copy(idx_hbm.at[wid], idx_v)

        @pl.loop(0, K)
        def _(kk):
            @pl.loop(0, EBLK, step=16)
            def _(j):
                src_v[kk, pl.ds(j, 16)] = idx_v[kk, pl.ds(j, 16)] & 0xFFFF

        for w in range(NWIN):
            wlo = w * WROWS
            rows_w = min(WROWS, N_PAD - wlo)
            rpw = rows_w // NS

            @pl.loop(0, K)
            def _(kk):
                @pl.loop(0, EBLK, step=16)
                def _(j):
                    lv = lax.shift_right_logical(idx_v[kk, pl.ds(j, 16)], 16) - wlo
                    ok = (lv >= 0) & (lv < rows_w)
                    dst_l[kk, pl.ds(j, 16)] = jnp.where(ok, lv, WROWS)

            _zero_fill(bufs, EBLK, D)
            for off in range(0, rpw, EBLK):
                sz = min(EBLK, rpw - off)
                pltpu.sync_copy(bufs.at[pl.ds(0, sz)],
                                acc.at[pl.ds(s * rpw + off, sz)])
            plsc.subcore_barrier()

            for b in range(NBUF):
                pltpu.async_copy(vals_hbm.at[src_v.at[b]],
                                 bufs.at[pl.ds(b * EBLK, EBLK)], sem_g)

            @pl.loop(0, K, step=NBUF)
            def _(kk):
                for b in range(NBUF):
                    bref = bufs.at[pl.ds(b * EBLK, EBLK)]
                    pltpu.make_async_copy(vals_hbm.at[src_v.at[kk + b]],
                                          bref, sem_g).wait()
                    pltpu.async_copy(bref, acc.at[dst_l.at[kk + b]], sem_s,
                                     add=True)
                for b in range(NBUF):
                    bref = bufs.at[pl.ds(b * EBLK, EBLK)]
                    pltpu.make_async_copy(bref, acc.at[dst_l.at[kk + b]],
                                          sem_s).wait()
                for b in range(NBUF):
                    bref = bufs.at[pl.ds(b * EBLK, EBLK)]

                    @pl.when(kk + NBUF + b < K)
                    def _():
                        pltpu.async_copy(vals_hbm.at[src_v.at[kk + NBUF + b]],
                                         bref, sem_g)

            plsc.subcore_barrier()
            pltpu.sync_copy(acc.at[pl.ds(s * rpw, rpw)],
                            out_hbm.at[c].at[pl.ds(wlo + s * rpw, rpw)])

    return k


def _sc_segment_sum(vals_pad, idx3):
    return _sc_segment_sum_kernel()(vals_pad, idx3)


def _sc_degree(idx4):
    @functools.partial(
        pl.kernel,
        out_type=jax.ShapeDtypeStruct((NW, N_PAD), jnp.float32),
        mesh=_mesh(),
        scratch_types=[
            pltpu.VMEM((KD, EBLK), jnp.int32),
            pltpu.VMEM((N_PAD,), jnp.float32),
        ],
        compiler_params=_sc_params(),
    )
    def k(idx_hbm, out_hbm, idx_v, hist):
        c = lax.axis_index("c")
        s = lax.axis_index("s")
        wid = s * NC + c
        pltpu.sync_copy(idx_hbm.at[wid], idx_v)
        zvec = jnp.zeros((16,), jnp.float32)

        @pl.loop(0, N_PAD, step=16)
        def _(j):
            hist[pl.ds(j, 16)] = zvec

        ones = jnp.ones((16,), jnp.float32)

        @pl.loop(0, KD)
        def _(kk):
            @pl.loop(0, EBLK, step=16)
            def _(j):
                dv = lax.shift_right_logical(idx_v[kk, pl.ds(j, 16)], 16)
                plsc.addupdate_scatter(hist, [dv], ones)

        pltpu.sync_copy(hist, out_hbm.at[wid])

    return k(idx4)


def _ln(v, s, b):
    m = jnp.mean(v, axis=-1, keepdims=True)
    c = v - m
    var = jnp.mean(c * c, axis=-1, keepdims=True)
    return c * lax.rsqrt(var + 1e-5) * s + b


def _dot(x, w):
    return jnp.dot(x, w, preferred_element_type=jnp.float32,
                   precision=lax.Precision.HIGHEST)


def _a_from_deg(deg_ref):
    deg = jnp.sum(deg_ref[...], axis=0)[:, None] + 1.0
    return lax.rsqrt(deg)


def _row_mask(i):
    rows = i * BR + lax.broadcasted_iota(jnp.int32, (BR, 1), 0)
    return rows < N


_W_SPEC = lambda r, c: pl.BlockSpec((r, c), lambda i: (0, 0))
_DEG_SPEC = pl.BlockSpec((NW, BR), lambda i: (0, i))
_ROW_SPEC = lambda w: pl.BlockSpec((BR, w), lambda i: (i, 0))
_PART_SPEC = pl.BlockSpec((NC, BR, D), lambda i: (0, i, 0))


def _tc_prep(x_pad, deg_parts, wg):
    def body(x_ref, deg_ref, wg_ref, g_ref):
        i = pl.program_id(0)
        a = _a_from_deg(deg_ref)
        g = _dot(x_ref[...], wg_ref[...]) * a
        g_ref[...] = jnp.where(_row_mask(i), g, 0.0)

    return pl.pallas_call(
        body,
        grid=(N_PAD // BR,),
        in_specs=[_ROW_SPEC(D), _DEG_SPEC, _W_SPEC(D, D)],
        out_specs=_ROW_SPEC(D),
        out_shape=jax.ShapeDtypeStruct((N_PAD, D), jnp.float32),
    )(x_pad, deg_parts, wg)


def _tc_layer(h_pad, g_pad, s_parts, deg_parts, p, wg_next, woff, boff):
    last = woff is not None

    def body(h_ref, g_ref, s_ref, deg_ref, wg2_ref,
             bg_ref, l1s_ref, l1b_ref, wf1_ref, bf1_ref, wf2_ref, bf2_ref,
             l2s_ref, l2b_ref, boff_ref, h2_ref, aux_ref):
        i = pl.program_id(0)
        a = _a_from_deg(deg_ref)
        h = h_ref[...]
        h_local = a * (s_ref[0] + s_ref[1] + g_ref[...]) + bg_ref[...]
        t = _ln(h + h_local, l1s_ref[...], l1b_ref[...])
        u = jnp.maximum(_dot(t, wf1_ref[...]) + bf1_ref[...], 0.0)
        ff = _dot(u, wf2_ref[...]) + bf2_ref[...]
        h2 = _ln(t + ff, l2s_ref[...], l2b_ref[...])
        h2_ref[...] = h2
        if last:
            aux_ref[...] = _dot(h + h2, wg2_ref[...]) + 2.0 * boff_ref[...]
        else:
            g2 = _dot(h2, wg2_ref[...]) * a
            aux_ref[...] = jnp.where(_row_mask(i), g2, 0.0)

    aux_w = OFF if last else D
    w2 = woff if last else wg_next
    return pl.pallas_call(
        body,
        grid=(N_PAD // BR,),
        in_specs=[
            _ROW_SPEC(D), _ROW_SPEC(D), _PART_SPEC, _DEG_SPEC,
            _W_SPEC(D, aux_w),
            _W_SPEC(1, D), _W_SPEC(1, D), _W_SPEC(1, D),
            _W_SPEC(D, DFF), _W_SPEC(1, DFF), _W_SPEC(DFF, D), _W_SPEC(1, D),
            _W_SPEC(1, D), _W_SPEC(1, D), _W_SPEC(1, aux_w),
        ],
        out_specs=[_ROW_SPEC(D), _ROW_SPEC(aux_w)],
        out_shape=[
            jax.ShapeDtypeStruct((N_PAD, D), jnp.float32),
            jax.ShapeDtypeStruct((N_PAD, aux_w), jnp.float32),
        ],
    )(h_pad, g_pad, s_parts, deg_parts, w2,
      p['bg'].reshape(1, D), p['ln1s'].reshape(1, D), p['ln1b'].reshape(1, D),
      p['Wf1'], p['bf1'].reshape(1, DFF), p['Wf2'], p['bf2'].reshape(1, D),
      p['ln2s'].reshape(1, D), p['ln2b'].reshape(1, D),
      (boff if last else jnp.zeros((aux_w,), jnp.float32)).reshape(1, aux_w))


def kernel(x, edge_index, node_indices, params):
    src = edge_index[0]
    dst = edge_index[1]
    pad = jnp.full((E_PAD - E,), DUMMY, jnp.int32)
    srcp = jnp.concatenate([src, pad])
    dstp = jnp.concatenate([dst, pad])
    idx3 = ((dstp << 16) | srcp).reshape(NW, K, EBLK)
    x_pad = jnp.pad(x, ((0, N_PAD - N), (0, 0)))

    p0 = params['layer0']
    p1 = params['layer1']

    deg_full = _sc_degree(idx3)
    g0 = _tc_prep(x_pad, deg_full, p0['Wg'])
    s0 = _sc_segment_sum(g0, idx3)
    h1, g1 = _tc_layer(x_pad, g0, s0, deg_full, p0,
                       wg_next=p1['Wg'], woff=None, boff=None)
    s1 = _sc_segment_sum(g1, idx3)
    h2, pz = _tc_layer(h1, g1, s1, deg_full, p1,
                       wg_next=None, woff=params['Woff'], boff=params['boff'])
    return h2[:N], pz[:N]


# speedup vs baseline: 1.6389x; 1.6389x over previous
import dataclasses
import functools

import jax
import jax.numpy as jnp
from jax import lax
from jax.experimental import pallas as pl
from jax.experimental.pallas import tpu as pltpu
from jax.experimental.pallas import tpu_sc as plsc

N = 10000
D = 128
DFF = 256
OFF = 36
E = 320000

NC = 2
NS = 16
NW = NC * NS
EBLK = 128
K = 80
KD = K
E_PAD = NW * K * EBLK
N_PAD = 10240
NWIN = 3
WROWS = 3456
DUMMY = N

BR = 512


def _mesh():
    return plsc.VectorSubcoreMesh(core_axis_name="c", subcore_axis_name="s")


def _sc_params():
    cp = pltpu.CompilerParams()
    if "needs_layout_passes" in pltpu.CompilerParams.__dataclass_fields__:
        cp = dataclasses.replace(cp, needs_layout_passes=False)
    return cp


def _zero_fill(buf, rows, width, value=0.0):
    vec = jnp.full((16,), value, jnp.float32)

    @pl.loop(0, rows)
    def _(r):
        @pl.loop(0, width, step=16)
        def _(j):
            buf[r, pl.ds(j, 16)] = vec


@functools.cache
def _sc_segment_sum_kernel():
    @functools.partial(
        pl.kernel,
        out_type=jax.ShapeDtypeStruct((NC, N_PAD, D), jnp.float32),
        mesh=_mesh(),
        scratch_types=[
            pltpu.VMEM((K, EBLK), jnp.int32),
            pltpu.VMEM((K, EBLK), jnp.int32),
            pltpu.VMEM((K, EBLK), jnp.int32),
            pltpu.VMEM((K, EBLK), jnp.int32),
            pltpu.VMEM((EBLK, D), jnp.float32),
            pltpu.VMEM((EBLK, D), jnp.float32),
            pltpu.VMEM_SHARED((WROWS + 8, D), jnp.float32),
            pltpu.SMEM((NWIN + 1,), jnp.int32),
            pltpu.SemaphoreType.DMA,
            pltpu.SemaphoreType.DMA,
        ],
        compiler_params=_sc_params(),
    )
    def k(vals_hbm, idx_hbm, out_hbm,
          idx_v, sbuf, dbuf, dst_l, buf_a, buf_b, acc, off_ref, sem_g, sem_s):
        c = lax.axis_index("c")
        s = lax.axis_index("s")
        wid = s * NC + c
        pltpu.sync_copy(idx_hbm.at[wid], idx_v)

        for w in range(NWIN):
            wlo = w * WROWS
            rows_w = min(WROWS, N_PAD - wlo)
            rpw = rows_w // NS

            off_ref[w] = 0
            dummy_src = jnp.full((16,), DUMMY, jnp.int32)
            dummy_dst = jnp.full((16,), WROWS, jnp.int32)

            @pl.loop(0, K)
            def _(kk):
                @pl.loop(0, EBLK, step=16)
                def _(j):
                    sbuf[kk, pl.ds(j, 16)] = dummy_src
                    dbuf[kk, pl.ds(j, 16)] = dummy_dst

            @pl.loop(0, K)
            def _(kk):
                @pl.loop(0, EBLK, step=16)
                def _(j):
                    v = idx_v[kk, pl.ds(j, 16)]
                    src16 = v & 0xFFFF
                    lv = lax.shift_right_logical(v, 16) - wlo
                    m = (lv >= 0) & (lv < rows_w)
                    mi = m.astype(jnp.int32)
                    offw = off_ref[w]
                    pos = offw + plsc.cumsum(mi) - 1
                    pos = jnp.where(m, pos, 0)
                    row = lax.shift_right_logical(pos, 7)
                    col = pos & (EBLK - 1)
                    plsc.store_scatter(sbuf, [row, col], src16, mask=m)
                    plsc.store_scatter(dbuf, [row, col], lv, mask=m)
                    off_ref[w] = offw + jnp.sum(mi)

            nblk = (off_ref[w] + (EBLK - 1)) // EBLK

            _zero_fill(buf_a, EBLK, D)
            for off in range(0, rpw, EBLK):
                sz = min(EBLK, rpw - off)
                pltpu.sync_copy(buf_a.at[pl.ds(0, sz)],
                                acc.at[pl.ds(s * rpw + off, sz)])
            plsc.subcore_barrier()

            @pl.loop(0, K)
            def _(kk):
                @pl.when(kk < nblk)
                def _():
                    pltpu.async_copy(vals_hbm.at[sbuf.at[kk]], buf_a, sem_g).wait()
                    pltpu.sync_copy(buf_a, acc.at[dbuf.at[kk]], add=True)

            plsc.subcore_barrier()
            pltpu.sync_copy(acc.at[pl.ds(s * rpw, rpw)],
                            out_hbm.at[c].at[pl.ds(wlo + s * rpw, rpw)])

    return k


def _sc_segment_sum(vals_pad, idx3):
    return _sc_segment_sum_kernel()(vals_pad, idx3)


def _sc_degree(idx4):
    @functools.partial(
        pl.kernel,
        out_type=jax.ShapeDtypeStruct((NW, N_PAD), jnp.float32),
        mesh=_mesh(),
        scratch_types=[
            pltpu.VMEM((KD, EBLK), jnp.int32),
            pltpu.VMEM((N_PAD,), jnp.float32),
        ],
        compiler_params=_sc_params(),
    )
    def k(idx_hbm, out_hbm, idx_v, hist):
        c = lax.axis_index("c")
        s = lax.axis_index("s")
        wid = s * NC + c
        pltpu.sync_copy(idx_hbm.at[wid], idx_v)
        zvec = jnp.zeros((16,), jnp.float32)

        @pl.loop(0, N_PAD, step=16)
        def _(j):
            hist[pl.ds(j, 16)] = zvec

        ones = jnp.ones((16,), jnp.float32)

        @pl.loop(0, KD)
        def _(kk):
            @pl.loop(0, EBLK, step=16)
            def _(j):
                dv = lax.shift_right_logical(idx_v[kk, pl.ds(j, 16)], 16)
                plsc.addupdate_scatter(hist, [dv], ones)

        pltpu.sync_copy(hist, out_hbm.at[wid])

    return k(idx4)


def _ln(v, s, b):
    m = jnp.mean(v, axis=-1, keepdims=True)
    c = v - m
    var = jnp.mean(c * c, axis=-1, keepdims=True)
    return c * lax.rsqrt(var + 1e-5) * s + b


def _dot(x, w):
    return jnp.dot(x, w, preferred_element_type=jnp.float32,
                   precision=lax.Precision.HIGHEST)


def _a_from_deg(deg_ref):
    deg = jnp.sum(deg_ref[...], axis=0)[:, None] + 1.0
    return lax.rsqrt(deg)


def _row_mask(i):
    rows = i * BR + lax.broadcasted_iota(jnp.int32, (BR, 1), 0)
    return rows < N


_W_SPEC = lambda r, c: pl.BlockSpec((r, c), lambda i: (0, 0))
_DEG_SPEC = pl.BlockSpec((NW, BR), lambda i: (0, i))
_ROW_SPEC = lambda w: pl.BlockSpec((BR, w), lambda i: (i, 0))
_PART_SPEC = pl.BlockSpec((NC, BR, D), lambda i: (0, i, 0))


def _tc_prep(x_pad, deg_parts, wg):
    def body(x_ref, deg_ref, wg_ref, g_ref):
        i = pl.program_id(0)
        a = _a_from_deg(deg_ref)
        g = _dot(x_ref[...], wg_ref[...]) * a
        g_ref[...] = jnp.where(_row_mask(i), g, 0.0)

    return pl.pallas_call(
        body,
        grid=(N_PAD // BR,),
        in_specs=[_ROW_SPEC(D), _DEG_SPEC, _W_SPEC(D, D)],
        out_specs=_ROW_SPEC(D),
        out_shape=jax.ShapeDtypeStruct((N_PAD, D), jnp.float32),
    )(x_pad, deg_parts, wg)


def _tc_layer(h_pad, g_pad, s_parts, deg_parts, p, wg_next, woff, boff):
    last = woff is not None

    def body(h_ref, g_ref, s_ref, deg_ref, wg2_ref,
             bg_ref, l1s_ref, l1b_ref, wf1_ref, bf1_ref, wf2_ref, bf2_ref,
             l2s_ref, l2b_ref, boff_ref, h2_ref, aux_ref):
        i = pl.program_id(0)
        a = _a_from_deg(deg_ref)
        h = h_ref[...]
        h_local = a * (s_ref[0] + s_ref[1] + g_ref[...]) + bg_ref[...]
        t = _ln(h + h_local, l1s_ref[...], l1b_ref[...])
        u = jnp.maximum(_dot(t, wf1_ref[...]) + bf1_ref[...], 0.0)
        ff = _dot(u, wf2_ref[...]) + bf2_ref[...]
        h2 = _ln(t + ff, l2s_ref[...], l2b_ref[...])
        h2_ref[...] = h2
        if last:
            aux_ref[...] = _dot(h + h2, wg2_ref[...]) + 2.0 * boff_ref[...]
        else:
            g2 = _dot(h2, wg2_ref[...]) * a
            aux_ref[...] = jnp.where(_row_mask(i), g2, 0.0)

    aux_w = OFF if last else D
    w2 = woff if last else wg_next
    return pl.pallas_call(
        body,
        grid=(N_PAD // BR,),
        in_specs=[
            _ROW_SPEC(D), _ROW_SPEC(D), _PART_SPEC, _DEG_SPEC,
            _W_SPEC(D, aux_w),
            _W_SPEC(1, D), _W_SPEC(1, D), _W_SPEC(1, D),
            _W_SPEC(D, DFF), _W_SPEC(1, DFF), _W_SPEC(DFF, D), _W_SPEC(1, D),
            _W_SPEC(1, D), _W_SPEC(1, D), _W_SPEC(1, aux_w),
        ],
        out_specs=[_ROW_SPEC(D), _ROW_SPEC(aux_w)],
        out_shape=[
            jax.ShapeDtypeStruct((N_PAD, D), jnp.float32),
            jax.ShapeDtypeStruct((N_PAD, aux_w), jnp.float32),
        ],
    )(h_pad, g_pad, s_parts, deg_parts, w2,
      p['bg'].reshape(1, D), p['ln1s'].reshape(1, D), p['ln1b'].reshape(1, D),
      p['Wf1'], p['bf1'].reshape(1, DFF), p['Wf2'], p['bf2'].reshape(1, D),
      p['ln2s'].reshape(1, D), p['ln2b'].reshape(1, D),
      (boff if last else jnp.zeros((aux_w,), jnp.float32)).reshape(1, aux_w))


def kernel(x, edge_index, node_indices, params):
    src = edge_index[0]
    dst = edge_index[1]
    pad = jnp.full((E_PAD - E,), DUMMY, jnp.int32)
    srcp = jnp.concatenate([src, pad])
    dstp = jnp.concatenate([dst, pad])
    idx3 = ((dstp << 16) | srcp).reshape(NW, K, EBLK)
    x_pad = jnp.pad(x, ((0, N_PAD - N), (0, 0)))

    p0 = params['layer0']
    p1 = params['layer1']

    deg_full = _sc_degree(idx3)
    g0 = _tc_prep(x_pad, deg_full, p0['Wg'])
    s0 = _sc_segment_sum(g0, idx3)
    h1, g1 = _tc_layer(x_pad, g0, s0, deg_full, p0,
                       wg_next=p1['Wg'], woff=None, boff=None)
    s1 = _sc_segment_sum(g1, idx3)
    h2, pz = _tc_layer(h1, g1, s1, deg_full, p1,
                       wg_next=None, woff=params['Woff'], boff=params['boff'])
    return h2[:N], pz[:N]


# speedup vs baseline: 1.8428x; 1.1244x over previous
import dataclasses
import functools

import jax
import jax.numpy as jnp
from jax import lax
from jax.experimental import pallas as pl
from jax.experimental.pallas import tpu as pltpu
from jax.experimental.pallas import tpu_sc as plsc

N = 10000
D = 128
DFF = 256
OFF = 36
E = 320000

NC = 2
NS = 16
NW = NC * NS
EBLK = 128
K = 80
KD = K
E_PAD = NW * K * EBLK
N_PAD = 10240
NWIN = 3
WROWS = 3456
DUMMY = N

BR = 512


def _mesh():
    return plsc.VectorSubcoreMesh(core_axis_name="c", subcore_axis_name="s")


def _sc_params():
    cp = pltpu.CompilerParams()
    if "needs_layout_passes" in pltpu.CompilerParams.__dataclass_fields__:
        cp = dataclasses.replace(cp, needs_layout_passes=False)
    return cp


def _zero_fill(buf, rows, width, value=0.0):
    vec = jnp.full((16,), value, jnp.float32)

    @pl.loop(0, rows)
    def _(r):
        @pl.loop(0, width, step=16)
        def _(j):
            buf[r, pl.ds(j, 16)] = vec


@functools.cache
def _sc_segment_sum_kernel():
    @functools.partial(
        pl.kernel,
        out_type=jax.ShapeDtypeStruct((NC, N_PAD, D), jnp.float32),
        mesh=_mesh(),
        scratch_types=[
            pltpu.VMEM((K, EBLK), jnp.int32),
            pltpu.VMEM((K, EBLK), jnp.int32),
            pltpu.VMEM((K, EBLK), jnp.int32),
            pltpu.VMEM((K, EBLK), jnp.int32),
            pltpu.VMEM((EBLK, D), jnp.float32),
            pltpu.VMEM((EBLK, D), jnp.float32),
            pltpu.VMEM_SHARED((WROWS + 8, D), jnp.float32),
            pltpu.SMEM((NWIN + 1,), jnp.int32),
            pltpu.SemaphoreType.DMA,
            pltpu.SemaphoreType.DMA,
        ],
        compiler_params=_sc_params(),
    )
    def k(vals_hbm, idx_hbm, out_hbm,
          idx_v, sbuf, dbuf, dst_l, buf_a, buf_b, acc, off_ref, sem_g, sem_s):
        c = lax.axis_index("c")
        s = lax.axis_index("s")
        wid = s * NC + c
        pltpu.sync_copy(idx_hbm.at[wid], idx_v)

        for w in range(NWIN):
            wlo = w * WROWS
            rows_w = min(WROWS, N_PAD - wlo)
            rpw = rows_w // NS

            off_ref[w] = 0
            dummy_src = jnp.full((16,), DUMMY, jnp.int32)
            dummy_dst = jnp.full((16,), WROWS, jnp.int32)

            @pl.loop(0, K)
            def _(kk):
                @pl.loop(0, EBLK, step=16)
                def _(j):
                    sbuf[kk, pl.ds(j, 16)] = dummy_src
                    dbuf[kk, pl.ds(j, 16)] = dummy_dst

            @pl.loop(0, K)
            def _(kk):
                @pl.loop(0, EBLK, step=16)
                def _(j):
                    v = idx_v[kk, pl.ds(j, 16)]
                    src16 = v & 0xFFFF
                    lv = lax.shift_right_logical(v, 16) - wlo
                    m = (lv >= 0) & (lv < rows_w)
                    mi = m.astype(jnp.int32)
                    offw = off_ref[w]
                    pos = offw + plsc.cumsum(mi) - 1
                    pos = jnp.where(m, pos, 0)
                    row = lax.shift_right_logical(pos, 7)
                    col = pos & (EBLK - 1)
                    plsc.store_scatter(sbuf, [row, col], src16, mask=m)
                    plsc.store_scatter(dbuf, [row, col], lv, mask=m)
                    off_ref[w] = offw + jnp.sum(mi)

            nblk = (off_ref[w] + (EBLK - 1)) // EBLK

            _zero_fill(buf_a, EBLK, D)
            for off in range(0, rpw, EBLK):
                sz = min(EBLK, rpw - off)
                pltpu.sync_copy(buf_a.at[pl.ds(0, sz)],
                                acc.at[pl.ds(s * rpw + off, sz)])
            plsc.subcore_barrier()

            @pl.when(nblk > 0)
            def _():
                pltpu.async_copy(vals_hbm.at[sbuf.at[0]], buf_a, sem_g)

            @pl.loop(0, K, step=2)
            def _(kk):
                @pl.when(kk < nblk)
                def _():
                    @pl.when(kk + 1 < nblk)
                    def _():
                        pltpu.async_copy(vals_hbm.at[sbuf.at[kk + 1]], buf_b, sem_s)

                    pltpu.make_async_copy(vals_hbm.at[sbuf.at[kk]], buf_a, sem_g).wait()
                    pltpu.sync_copy(buf_a, acc.at[dbuf.at[kk]], add=True)

                    @pl.when(kk + 2 < nblk)
                    def _():
                        pltpu.async_copy(vals_hbm.at[sbuf.at[kk + 2]], buf_a, sem_g)

                    @pl.when(kk + 1 < nblk)
                    def _():
                        pltpu.make_async_copy(vals_hbm.at[sbuf.at[kk + 1]], buf_b, sem_s).wait()
                        pltpu.sync_copy(buf_b, acc.at[dbuf.at[kk + 1]], add=True)

            plsc.subcore_barrier()
            pltpu.sync_copy(acc.at[pl.ds(s * rpw, rpw)],
                            out_hbm.at[c].at[pl.ds(wlo + s * rpw, rpw)])

    return k


def _sc_segment_sum(vals_pad, idx3):
    return _sc_segment_sum_kernel()(vals_pad, idx3)


def _sc_degree(idx4):
    @functools.partial(
        pl.kernel,
        out_type=jax.ShapeDtypeStruct((NW, N_PAD), jnp.float32),
        mesh=_mesh(),
        scratch_types=[
            pltpu.VMEM((KD, EBLK), jnp.int32),
            pltpu.VMEM((N_PAD,), jnp.float32),
        ],
        compiler_params=_sc_params(),
    )
    def k(idx_hbm, out_hbm, idx_v, hist):
        c = lax.axis_index("c")
        s = lax.axis_index("s")
        wid = s * NC + c
        pltpu.sync_copy(idx_hbm.at[wid], idx_v)
        zvec = jnp.zeros((16,), jnp.float32)

        @pl.loop(0, N_PAD, step=16)
        def _(j):
            hist[pl.ds(j, 16)] = zvec

        ones = jnp.ones((16,), jnp.float32)

        @pl.loop(0, KD)
        def _(kk):
            @pl.loop(0, EBLK, step=16)
            def _(j):
                dv = lax.shift_right_logical(idx_v[kk, pl.ds(j, 16)], 16)
                plsc.addupdate_scatter(hist, [dv], ones)

        pltpu.sync_copy(hist, out_hbm.at[wid])

    return k(idx4)


def _ln(v, s, b):
    m = jnp.mean(v, axis=-1, keepdims=True)
    c = v - m
    var = jnp.mean(c * c, axis=-1, keepdims=True)
    return c * lax.rsqrt(var + 1e-5) * s + b


def _dot(x, w):
    return jnp.dot(x, w, preferred_element_type=jnp.float32,
                   precision=lax.Precision.HIGHEST)


def _a_from_deg(deg_ref):
    deg = jnp.sum(deg_ref[...], axis=0)[:, None] + 1.0
    return lax.rsqrt(deg)


def _row_mask(i):
    rows = i * BR + lax.broadcasted_iota(jnp.int32, (BR, 1), 0)
    return rows < N


_W_SPEC = lambda r, c: pl.BlockSpec((r, c), lambda i: (0, 0))
_DEG_SPEC = pl.BlockSpec((NW, BR), lambda i: (0, i))
_ROW_SPEC = lambda w: pl.BlockSpec((BR, w), lambda i: (i, 0))
_PART_SPEC = pl.BlockSpec((NC, BR, D), lambda i: (0, i, 0))


def _tc_prep(x_pad, deg_parts, wg):
    def body(x_ref, deg_ref, wg_ref, g_ref):
        i = pl.program_id(0)
        a = _a_from_deg(deg_ref)
        g = _dot(x_ref[...], wg_ref[...]) * a
        g_ref[...] = jnp.where(_row_mask(i), g, 0.0)

    return pl.pallas_call(
        body,
        grid=(N_PAD // BR,),
        in_specs=[_ROW_SPEC(D), _DEG_SPEC, _W_SPEC(D, D)],
        out_specs=_ROW_SPEC(D),
        out_shape=jax.ShapeDtypeStruct((N_PAD, D), jnp.float32),
    )(x_pad, deg_parts, wg)


def _tc_layer(h_pad, g_pad, s_parts, deg_parts, p, wg_next, woff, boff):
    last = woff is not None

    def body(h_ref, g_ref, s_ref, deg_ref, wg2_ref,
             bg_ref, l1s_ref, l1b_ref, wf1_ref, bf1_ref, wf2_ref, bf2_ref,
             l2s_ref, l2b_ref, boff_ref, h2_ref, aux_ref):
        i = pl.program_id(0)
        a = _a_from_deg(deg_ref)
        h = h_ref[...]
        h_local = a * (s_ref[0] + s_ref[1] + g_ref[...]) + bg_ref[...]
        t = _ln(h + h_local, l1s_ref[...], l1b_ref[...])
        u = jnp.maximum(_dot(t, wf1_ref[...]) + bf1_ref[...], 0.0)
        ff = _dot(u, wf2_ref[...]) + bf2_ref[...]
        h2 = _ln(t + ff, l2s_ref[...], l2b_ref[...])
        h2_ref[...] = h2
        if last:
            aux_ref[...] = _dot(h + h2, wg2_ref[...]) + 2.0 * boff_ref[...]
        else:
            g2 = _dot(h2, wg2_ref[...]) * a
            aux_ref[...] = jnp.where(_row_mask(i), g2, 0.0)

    aux_w = OFF if last else D
    w2 = woff if last else wg_next
    return pl.pallas_call(
        body,
        grid=(N_PAD // BR,),
        in_specs=[
            _ROW_SPEC(D), _ROW_SPEC(D), _PART_SPEC, _DEG_SPEC,
            _W_SPEC(D, aux_w),
            _W_SPEC(1, D), _W_SPEC(1, D), _W_SPEC(1, D),
            _W_SPEC(D, DFF), _W_SPEC(1, DFF), _W_SPEC(DFF, D), _W_SPEC(1, D),
            _W_SPEC(1, D), _W_SPEC(1, D), _W_SPEC(1, aux_w),
        ],
        out_specs=[_ROW_SPEC(D), _ROW_SPEC(aux_w)],
        out_shape=[
            jax.ShapeDtypeStruct((N_PAD, D), jnp.float32),
            jax.ShapeDtypeStruct((N_PAD, aux_w), jnp.float32),
        ],
    )(h_pad, g_pad, s_parts, deg_parts, w2,
      p['bg'].reshape(1, D), p['ln1s'].reshape(1, D), p['ln1b'].reshape(1, D),
      p['Wf1'], p['bf1'].reshape(1, DFF), p['Wf2'], p['bf2'].reshape(1, D),
      p['ln2s'].reshape(1, D), p['ln2b'].reshape(1, D),
      (boff if last else jnp.zeros((aux_w,), jnp.float32)).reshape(1, aux_w))


def kernel(x, edge_index, node_indices, params):
    src = edge_index[0]
    dst = edge_index[1]
    pad = jnp.full((E_PAD - E,), DUMMY, jnp.int32)
    srcp = jnp.concatenate([src, pad])
    dstp = jnp.concatenate([dst, pad])
    idx3 = ((dstp << 16) | srcp).reshape(NW, K, EBLK)
    x_pad = jnp.pad(x, ((0, N_PAD - N), (0, 0)))

    p0 = params['layer0']
    p1 = params['layer1']

    deg_full = _sc_degree(idx3)
    g0 = _tc_prep(x_pad, deg_full, p0['Wg'])
    s0 = _sc_segment_sum(g0, idx3)
    h1, g1 = _tc_layer(x_pad, g0, s0, deg_full, p0,
                       wg_next=p1['Wg'], woff=None, boff=None)
    s1 = _sc_segment_sum(g1, idx3)
    h2, pz = _tc_layer(h1, g1, s1, deg_full, p1,
                       wg_next=None, woff=params['Woff'], boff=params['boff'])
    return h2[:N], pz[:N]


# speedup vs baseline: 1.9013x; 1.0318x over previous
import dataclasses
import functools

import jax
import jax.numpy as jnp
from jax import lax
from jax.experimental import pallas as pl
from jax.experimental.pallas import tpu as pltpu
from jax.experimental.pallas import tpu_sc as plsc

N = 10000
D = 128
DFF = 256
OFF = 36
E = 320000

NC = 2
NS = 16
NW = NC * NS
EBLK = 128
K = 80
KD = K
E_PAD = NW * K * EBLK
N_PAD = 10240
NWIN = 3
WROWS = 3456
DUMMY = N

BR = 1024


def _mesh():
    return plsc.VectorSubcoreMesh(core_axis_name="c", subcore_axis_name="s")


def _sc_params():
    cp = pltpu.CompilerParams()
    if "needs_layout_passes" in pltpu.CompilerParams.__dataclass_fields__:
        cp = dataclasses.replace(cp, needs_layout_passes=False)
    return cp


def _zero_fill(buf, rows, width, value=0.0):
    vec = jnp.full((16,), value, jnp.float32)

    @pl.loop(0, rows)
    def _(r):
        @pl.loop(0, width, step=16)
        def _(j):
            buf[r, pl.ds(j, 16)] = vec


@functools.cache
def _sc_segment_sum_kernel():
    @functools.partial(
        pl.kernel,
        out_type=jax.ShapeDtypeStruct((NC, N_PAD, D), jnp.float32),
        mesh=_mesh(),
        scratch_types=[
            pltpu.VMEM((K, EBLK), jnp.int32),
            pltpu.VMEM((K, EBLK), jnp.int32),
            pltpu.VMEM((K, EBLK), jnp.int32),
            pltpu.VMEM((K, EBLK), jnp.int32),
            pltpu.VMEM((K, EBLK), jnp.int32),
            pltpu.VMEM((K, EBLK), jnp.int32),
            pltpu.VMEM((EBLK, D), jnp.float32),
            pltpu.VMEM((EBLK, D), jnp.float32),
            pltpu.VMEM_SHARED((WROWS + 8, D), jnp.float32),
            pltpu.SMEM((NWIN + 1,), jnp.int32),
            pltpu.SemaphoreType.DMA,
            pltpu.SemaphoreType.DMA,
        ],
        compiler_params=_sc_params(),
    )
    def k(vals_hbm, idx_hbm, out_hbm,
          idx_v, sb0, sb1, db0, db1, db2, buf_a, buf_b, acc, off_ref, sem_g, sem_s):
        c = lax.axis_index("c")
        s = lax.axis_index("s")
        wid = s * NC + c
        pltpu.sync_copy(idx_hbm.at[wid], idx_v)
        sbufs = [sb0, sb1, idx_v]
        dbufs = [db0, db1, db2]
        dummy_src = jnp.full((16,), DUMMY, jnp.int32)
        dummy_dst = jnp.full((16,), WROWS, jnp.int32)

        for w in range(NWIN):
            off_ref[w] = 0

        @pl.loop(0, K)
        def _(kk):
            @pl.loop(0, EBLK, step=16)
            def _(j):
                for w in range(NWIN):
                    if w < 2:
                        sbufs[w][kk, pl.ds(j, 16)] = dummy_src
                    dbufs[w][kk, pl.ds(j, 16)] = dummy_dst

        @pl.loop(0, K)
        def _(kk):
            @pl.loop(0, EBLK, step=16)
            def _(j):
                v = idx_v[kk, pl.ds(j, 16)]
                src16 = v & 0xFFFF
                d16 = lax.shift_right_logical(v, 16)
                for w in range(NWIN):
                    wlo = w * WROWS
                    rows_w = min(WROWS, N_PAD - wlo)
                    lv = d16 - wlo
                    m = (lv >= 0) & (lv < rows_w)
                    mi = m.astype(jnp.int32)
                    offw = off_ref[w]
                    pos = offw + plsc.cumsum(mi) - 1
                    pos = jnp.where(m, pos, 0)
                    row = lax.shift_right_logical(pos, 7)
                    col = pos & (EBLK - 1)
                    plsc.store_scatter(sbufs[w], [row, col], src16, mask=m)
                    plsc.store_scatter(dbufs[w], [row, col], lv, mask=m)
                    off_ref[w] = offw + jnp.sum(mi)

        off2 = off_ref[2]

        @pl.when(off2 < K * EBLK)
        def _():
            kk_last = lax.shift_right_logical(off2, 7)

            @pl.loop(0, EBLK, step=16)
            def _(j):
                slot = kk_last * EBLK + j + lax.iota(jnp.int32, 16)
                keep = slot < off2
                cur = idx_v[kk_last, pl.ds(j, 16)]
                idx_v[kk_last, pl.ds(j, 16)] = jnp.where(keep, cur, DUMMY)

        for w in range(NWIN):
            wlo = w * WROWS
            rows_w = min(WROWS, N_PAD - wlo)
            rpw = rows_w // NS
            sbuf = sbufs[w]
            dbuf = dbufs[w]
            nblk = (off_ref[w] + (EBLK - 1)) // EBLK

            _zero_fill(buf_a, EBLK, D)
            for off in range(0, rpw, EBLK):
                sz = min(EBLK, rpw - off)
                pltpu.sync_copy(buf_a.at[pl.ds(0, sz)],
                                acc.at[pl.ds(s * rpw + off, sz)])
            plsc.subcore_barrier()

            @pl.when(nblk > 0)
            def _():
                pltpu.async_copy(vals_hbm.at[sbuf.at[0]], buf_a, sem_g)

            @pl.loop(0, K, step=2)
            def _(kk):
                @pl.when(kk < nblk)
                def _():
                    @pl.when(kk + 1 < nblk)
                    def _():
                        pltpu.async_copy(vals_hbm.at[sbuf.at[kk + 1]], buf_b, sem_s)

                    pltpu.make_async_copy(vals_hbm.at[sbuf.at[kk]], buf_a, sem_g).wait()
                    pltpu.sync_copy(buf_a, acc.at[dbuf.at[kk]], add=True)

                    @pl.when(kk + 2 < nblk)
                    def _():
                        pltpu.async_copy(vals_hbm.at[sbuf.at[kk + 2]], buf_a, sem_g)

                    @pl.when(kk + 1 < nblk)
                    def _():
                        pltpu.make_async_copy(vals_hbm.at[sbuf.at[kk + 1]], buf_b, sem_s).wait()
                        pltpu.sync_copy(buf_b, acc.at[dbuf.at[kk + 1]], add=True)

            plsc.subcore_barrier()
            pltpu.sync_copy(acc.at[pl.ds(s * rpw, rpw)],
                            out_hbm.at[c].at[pl.ds(wlo + s * rpw, rpw)])

    return k


def _sc_segment_sum(vals_pad, idx3):
    return _sc_segment_sum_kernel()(vals_pad, idx3)


def _sc_degree(idx4):
    @functools.partial(
        pl.kernel,
        out_type=jax.ShapeDtypeStruct((NW, N_PAD), jnp.float32),
        mesh=_mesh(),
        scratch_types=[
            pltpu.VMEM((KD, EBLK), jnp.int32),
            pltpu.VMEM((N_PAD,), jnp.float32),
        ],
        compiler_params=_sc_params(),
    )
    def k(idx_hbm, out_hbm, idx_v, hist):
        c = lax.axis_index("c")
        s = lax.axis_index("s")
        wid = s * NC + c
        pltpu.sync_copy(idx_hbm.at[wid], idx_v)
        zvec = jnp.zeros((16,), jnp.float32)

        @pl.loop(0, N_PAD, step=16)
        def _(j):
            hist[pl.ds(j, 16)] = zvec

        ones = jnp.ones((16,), jnp.float32)

        @pl.loop(0, KD)
        def _(kk):
            @pl.loop(0, EBLK, step=16)
            def _(j):
                dv = lax.shift_right_logical(idx_v[kk, pl.ds(j, 16)], 16)
                plsc.addupdate_scatter(hist, [dv], ones)

        pltpu.sync_copy(hist, out_hbm.at[wid])

    return k(idx4)


def _ln(v, s, b):
    m = jnp.mean(v, axis=-1, keepdims=True)
    c = v - m
    var = jnp.mean(c * c, axis=-1, keepdims=True)
    return c * lax.rsqrt(var + 1e-5) * s + b


def _dot(x, w):
    return jnp.dot(x, w, preferred_element_type=jnp.float32,
                   precision=lax.Precision.HIGHEST)


def _a_from_deg(deg_ref):
    deg = jnp.sum(deg_ref[...], axis=0)[:, None] + 1.0
    return lax.rsqrt(deg)


def _row_mask(i):
    rows = i * BR + lax.broadcasted_iota(jnp.int32, (BR, 1), 0)
    return rows < N


_W_SPEC = lambda r, c: pl.BlockSpec((r, c), lambda i: (0, 0))
_DEG_SPEC = pl.BlockSpec((NW, BR), lambda i: (0, i))
_ROW_SPEC = lambda w: pl.BlockSpec((BR, w), lambda i: (i, 0))
_PART_SPEC = pl.BlockSpec((NC, BR, D), lambda i: (0, i, 0))


def _tc_prep(x_pad, deg_parts, wg):
    def body(x_ref, deg_ref, wg_ref, g_ref):
        i = pl.program_id(0)
        a = _a_from_deg(deg_ref)
        g = _dot(x_ref[...], wg_ref[...]) * a
        g_ref[...] = jnp.where(_row_mask(i), g, 0.0)

    return pl.pallas_call(
        body,
        grid=(N_PAD // BR,),
        in_specs=[_ROW_SPEC(D), _DEG_SPEC, _W_SPEC(D, D)],
        out_specs=_ROW_SPEC(D),
        out_shape=jax.ShapeDtypeStruct((N_PAD, D), jnp.float32),
    )(x_pad, deg_parts, wg)


def _tc_layer(h_pad, g_pad, s_parts, deg_parts, p, wg_next, woff, boff):
    last = woff is not None

    def body(h_ref, g_ref, s_ref, deg_ref, wg2_ref,
             bg_ref, l1s_ref, l1b_ref, wf1_ref, bf1_ref, wf2_ref, bf2_ref,
             l2s_ref, l2b_ref, boff_ref, h2_ref, aux_ref):
        i = pl.program_id(0)
        a = _a_from_deg(deg_ref)
        h = h_ref[...]
        h_local = a * (s_ref[0] + s_ref[1] + g_ref[...]) + bg_ref[...]
        t = _ln(h + h_local, l1s_ref[...], l1b_ref[...])
        u = jnp.maximum(_dot(t, wf1_ref[...]) + bf1_ref[...], 0.0)
        ff = _dot(u, wf2_ref[...]) + bf2_ref[...]
        h2 = _ln(t + ff, l2s_ref[...], l2b_ref[...])
        h2_ref[...] = h2
        if last:
            aux_ref[...] = _dot(h + h2, wg2_ref[...]) + 2.0 * boff_ref[...]
        else:
            g2 = _dot(h2, wg2_ref[...]) * a
            aux_ref[...] = jnp.where(_row_mask(i), g2, 0.0)

    aux_w = OFF if last else D
    w2 = woff if last else wg_next
    return pl.pallas_call(
        body,
        grid=(N_PAD // BR,),
        in_specs=[
            _ROW_SPEC(D), _ROW_SPEC(D), _PART_SPEC, _DEG_SPEC,
            _W_SPEC(D, aux_w),
            _W_SPEC(1, D), _W_SPEC(1, D), _W_SPEC(1, D),
            _W_SPEC(D, DFF), _W_SPEC(1, DFF), _W_SPEC(DFF, D), _W_SPEC(1, D),
            _W_SPEC(1, D), _W_SPEC(1, D), _W_SPEC(1, aux_w),
        ],
        out_specs=[_ROW_SPEC(D), _ROW_SPEC(aux_w)],
        out_shape=[
            jax.ShapeDtypeStruct((N_PAD, D), jnp.float32),
            jax.ShapeDtypeStruct((N_PAD, aux_w), jnp.float32),
        ],
    )(h_pad, g_pad, s_parts, deg_parts, w2,
      p['bg'].reshape(1, D), p['ln1s'].reshape(1, D), p['ln1b'].reshape(1, D),
      p['Wf1'], p['bf1'].reshape(1, DFF), p['Wf2'], p['bf2'].reshape(1, D),
      p['ln2s'].reshape(1, D), p['ln2b'].reshape(1, D),
      (boff if last else jnp.zeros((aux_w,), jnp.float32)).reshape(1, aux_w))


def kernel(x, edge_index, node_indices, params):
    src = edge_index[0]
    dst = edge_index[1]
    pad = jnp.full((E_PAD - E,), DUMMY, jnp.int32)
    srcp = jnp.concatenate([src, pad])
    dstp = jnp.concatenate([dst, pad])
    idx3 = ((dstp << 16) | srcp).reshape(NW, K, EBLK)
    x_pad = jnp.pad(x, ((0, N_PAD - N), (0, 0)))

    p0 = params['layer0']
    p1 = params['layer1']

    deg_full = _sc_degree(idx3)
    g0 = _tc_prep(x_pad, deg_full, p0['Wg'])
    s0 = _sc_segment_sum(g0, idx3)
    h1, g1 = _tc_layer(x_pad, g0, s0, deg_full, p0,
                       wg_next=p1['Wg'], woff=None, boff=None)
    s1 = _sc_segment_sum(g1, idx3)
    h2, pz = _tc_layer(h1, g1, s1, deg_full, p1,
                       wg_next=None, woff=params['Woff'], boff=params['boff'])
    return h2[:N], pz[:N]
